# baseline (speedup 1.0000x reference)
import functools

import jax
import jax.numpy as jnp
from jax.experimental import pallas as pl

HIDDEN = 2048
NUM_EXPERTS = 8
TOP_K = 2
BLOCK_ROWS = 2048
NSPLIT = 4
CHUNK = BLOCK_ROWS // NSPLIT


def _router_block(*refs):
    x_refs = refs[:NSPLIT]
    w_ref = refs[NSPLIT]
    probs_ref, idx_ref, wts_ref = refs[NSPLIT + 1:]
    w = w_ref[...]
    for j in range(NSPLIT):
        logits_t = jax.lax.dot_general(
            w, x_refs[j][...],
            dimension_numbers=(((1,), (1,)), ((), ())),
            preferred_element_type=jnp.float32,
        )
        m = jnp.max(logits_t, axis=0, keepdims=True)
        e = jnp.exp(logits_t - m)
        s = jnp.sum(e, axis=0, keepdims=True)
        probs_t = e / s

        iota = jax.lax.broadcasted_iota(jnp.int32, probs_t.shape, 0)
        v1 = jnp.max(probs_t, axis=0, keepdims=True)
        i1 = jnp.min(jnp.where(probs_t == v1, iota, NUM_EXPERTS), axis=0,
                     keepdims=True)
        masked = jnp.where(iota == i1, -jnp.inf, probs_t)
        v2 = jnp.max(masked, axis=0, keepdims=True)
        i2 = jnp.min(jnp.where(masked == v2, iota, NUM_EXPERTS), axis=0,
                     keepdims=True)

        rows = pl.ds(j * CHUNK, CHUNK)
        probs_ref[rows, :] = probs_t.T
        idx_ref[rows, :] = jnp.concatenate([i1, i2], axis=0).T
        denom = v1 + v2
        wts_ref[rows, :] = jnp.concatenate([v1 / denom, v2 / denom], axis=0).T


@functools.partial(jax.jit, static_argnames=("interpret",))
def kernel(hidden_states, gate_w, interpret=False):
    b, s, h = hidden_states.shape
    n = b * s
    x = hidden_states.reshape(n, h)

    grid = (n // BLOCK_ROWS,)
    in_specs = [
        pl.BlockSpec((CHUNK, h), lambda i, j=j: (NSPLIT * i + j, 0))
        for j in range(NSPLIT)
    ] + [pl.BlockSpec((NUM_EXPERTS, h), lambda i: (0, 0))]

    probs, idx, wts = pl.pallas_call(
        _router_block,
        grid=grid,
        in_specs=in_specs,
        out_specs=[
            pl.BlockSpec((BLOCK_ROWS, NUM_EXPERTS), lambda i: (i, 0)),
            pl.BlockSpec((BLOCK_ROWS, TOP_K), lambda i: (i, 0)),
            pl.BlockSpec((BLOCK_ROWS, TOP_K), lambda i: (i, 0)),
        ],
        out_shape=[
            jax.ShapeDtypeStruct((n, NUM_EXPERTS), jnp.float32),
            jax.ShapeDtypeStruct((n, TOP_K), jnp.int32),
            jax.ShapeDtypeStruct((n, TOP_K), jnp.float32),
        ],
        interpret=interpret,
    )(*([x] * NSPLIT), gate_w)

    return (
        probs.reshape(b, s, NUM_EXPERTS),
        idx.reshape(b, s, TOP_K),
        wts.reshape(b, s, TOP_K),
    )


# speedup vs baseline: 1.5347x; 1.5347x over previous
import functools

import jax
import jax.numpy as jnp
from jax.experimental import pallas as pl

HIDDEN = 2048
NUM_EXPERTS = 8
TOP_K = 2
BLOCK_ROWS = 2048
NSPLIT = 4
CHUNK = BLOCK_ROWS // NSPLIT


def _router_block(*refs):
    x_refs = refs[:NSPLIT]
    w_ref = refs[NSPLIT]
    probs_ref, idx_ref, wts_ref = refs[NSPLIT + 1:]
    w = w_ref[...]
    for j in range(NSPLIT):
        logits_t = jax.lax.dot_general(
            w, x_refs[j][...],
            dimension_numbers=(((1,), (1,)), ((), ())),
            preferred_element_type=jnp.float32,
        )
        m = jnp.max(logits_t, axis=0, keepdims=True)
        e = jnp.exp(logits_t - m)
        s = jnp.sum(e, axis=0, keepdims=True)
        probs_t = e / s

        iota = jax.lax.broadcasted_iota(jnp.int32, probs_t.shape, 0)
        v1 = jnp.max(probs_t, axis=0, keepdims=True)
        i1 = jnp.min(jnp.where(probs_t == v1, iota, NUM_EXPERTS), axis=0,
                     keepdims=True)
        masked = jnp.where(iota == i1, -jnp.inf, probs_t)
        v2 = jnp.max(masked, axis=0, keepdims=True)
        i2 = jnp.min(jnp.where(masked == v2, iota, NUM_EXPERTS), axis=0,
                     keepdims=True)

        cols = pl.ds(j * CHUNK, CHUNK)
        probs_ref[:, cols] = probs_t
        idx_ref[:, cols] = jnp.concatenate([i1, i2], axis=0)
        denom = v1 + v2
        wts_ref[:, cols] = jnp.concatenate([v1 / denom, v2 / denom], axis=0)


@functools.partial(jax.jit, static_argnames=("interpret",))
def kernel(hidden_states, gate_w, interpret=False):
    b, s, h = hidden_states.shape
    n = b * s
    x = hidden_states.reshape(n, h)

    grid = (n // BLOCK_ROWS,)
    in_specs = [
        pl.BlockSpec((CHUNK, h), lambda i, j=j: (NSPLIT * i + j, 0))
        for j in range(NSPLIT)
    ] + [pl.BlockSpec((NUM_EXPERTS, h), lambda i: (0, 0))]

    probs_t, idx_t, wts_t = pl.pallas_call(
        _router_block,
        grid=grid,
        in_specs=in_specs,
        out_specs=[
            pl.BlockSpec((NUM_EXPERTS, BLOCK_ROWS), lambda i: (0, i)),
            pl.BlockSpec((TOP_K, BLOCK_ROWS), lambda i: (0, i)),
            pl.BlockSpec((TOP_K, BLOCK_ROWS), lambda i: (0, i)),
        ],
        out_shape=[
            jax.ShapeDtypeStruct((NUM_EXPERTS, n), jnp.float32),
            jax.ShapeDtypeStruct((TOP_K, n), jnp.int32),
            jax.ShapeDtypeStruct((TOP_K, n), jnp.float32),
        ],
        interpret=interpret,
    )(*([x] * NSPLIT), gate_w)

    return (
        probs_t.T.reshape(b, s, NUM_EXPERTS),
        idx_t.T.reshape(b, s, TOP_K),
        wts_t.T.reshape(b, s, TOP_K),
    )
